# Initial kernel scaffold; baseline (speedup 1.0000x reference)
import jax
import jax.numpy as jnp
from jax.experimental import pallas as pl


def kernel(h, message, node_type_ids, W1, b1, W2, b2):
    raise NotImplementedError("write your pallas kernel here")


# speedup vs baseline: 1.9221x; 1.9221x over previous
import functools

import jax
import jax.numpy as jnp
from jax.experimental import pallas as pl
from jax.experimental.pallas import tpu as pltpu

N_TYPES = 8
D = 128
N = 100000
BLK = 800
N_BLOCKS = N // BLK


def _sn(W):
    R = W.shape[0]
    u = jnp.full((R, 1), 1.0 / jnp.sqrt(jnp.float32(R)), dtype=jnp.float32)
    v = None
    for _ in range(7):
        v = jnp.sum(W * u, axis=0, keepdims=True)
        v = v / (jnp.sqrt(jnp.sum(v * v)) + 1e-12)
        u = jnp.sum(W * v, axis=1, keepdims=True)
        u = u / (jnp.sqrt(jnp.sum(u * u)) + 1e-12)
    Wv = jnp.sum(W * v, axis=1, keepdims=True)
    sigma = jnp.sum(u * Wv)
    return W / sigma


def _norm_body(w1_ref, w2_ref, w1o_ref, w2o_ref):
    w1o_ref[0] = _sn(w1_ref[0])
    w2o_ref[0] = _sn(w2_ref[0])


def _moe_body(ids_ref, h_ref, m_ref, w1_ref, b1_ref, w2_ref, b2_ref, out_ref):
    hx = h_ref[...]
    mx = m_ref[...]
    ids = ids_ref[...]
    acc = jnp.zeros((BLK, D), jnp.float32)
    for t in range(N_TYPES):
        y = (
            jnp.dot(hx, w1_ref[t, :D, :], preferred_element_type=jnp.float32)
            + jnp.dot(mx, w1_ref[t, D:, :], preferred_element_type=jnp.float32)
            + b1_ref[t]
        )
        y = y * jax.nn.sigmoid(y)
        z = jnp.dot(y, w2_ref[t], preferred_element_type=jnp.float32) + b2_ref[t]
        acc = acc + z * (ids == t).astype(jnp.float32)
    out_ref[...] = acc


def kernel(h, message, node_type_ids, W1, b1, W2, b2, interpret=False):
    W1n, W2n = pl.pallas_call(
        _norm_body,
        grid=(N_TYPES,),
        in_specs=[
            pl.BlockSpec((1, 2 * D, D), lambda i: (i, 0, 0)),
            pl.BlockSpec((1, D, D), lambda i: (i, 0, 0)),
        ],
        out_specs=[
            pl.BlockSpec((1, 2 * D, D), lambda i: (i, 0, 0)),
            pl.BlockSpec((1, D, D), lambda i: (i, 0, 0)),
        ],
        out_shape=[
            jax.ShapeDtypeStruct((N_TYPES, 2 * D, D), jnp.float32),
            jax.ShapeDtypeStruct((N_TYPES, D, D), jnp.float32),
        ],
        interpret=interpret,
    )(W1, W2)

    ids3 = node_type_ids.astype(jnp.int32).reshape(N, 1)
    dh = pl.pallas_call(
        _moe_body,
        grid=(N_BLOCKS,),
        in_specs=[
            pl.BlockSpec((BLK, 1), lambda i: (i, 0)),
            pl.BlockSpec((BLK, D), lambda i: (i, 0)),
            pl.BlockSpec((BLK, D), lambda i: (i, 0)),
            pl.BlockSpec((N_TYPES, 2 * D, D), lambda i: (0, 0, 0)),
            pl.BlockSpec((N_TYPES, D), lambda i: (0, 0)),
            pl.BlockSpec((N_TYPES, D, D), lambda i: (0, 0, 0)),
            pl.BlockSpec((N_TYPES, D), lambda i: (0, 0)),
        ],
        out_specs=pl.BlockSpec((BLK, D), lambda i: (i, 0)),
        out_shape=jax.ShapeDtypeStruct((N, D), jnp.float32),
        compiler_params=pltpu.CompilerParams(
            dimension_semantics=("arbitrary",),
        ),
        interpret=interpret,
    )(ids3, h, message, W1n, b1, W2n, b2)
    return dh
